# baseline (speedup 1.0000x reference)
import functools

import jax
import jax.numpy as jnp
from jax import lax
from jax.experimental import pallas as pl
from jax.experimental.pallas import tpu as pltpu
from jax.experimental.pallas import tpu_sc as plsc

L = 16
CW = 512
RB = 64
FLUSH_AT = RB - L


def _lanes():
    return lax.iota(jnp.int32, L)


def _make_sweep_kernel(B, V, D, NC, NS):
    NCHUNK_FULL = V // CW
    TAIL = V - NCHUNK_FULL * CW
    NK = (NCHUNK_FULL + NS - 1) // NS
    NV = B // L
    RROWS = B + RB

    mesh = plsc.VectorSubcoreMesh(core_axis_name="c", subcore_axis_name="s")

    @functools.partial(
        pl.kernel,
        mesh=mesh,
        out_type=(jax.ShapeDtypeStruct((RROWS, 128), jnp.float32),
                  jax.ShapeDtypeStruct((RROWS, 128), jnp.float32)),
        scratch_types=[
            pltpu.VMEM((B // 2,), jnp.int32),
            pltpu.VMEM((B + 4 * L,), jnp.int32),
            pltpu.VMEM((B + L,), jnp.int32),
            pltpu.VMEM((128,), jnp.int32),
            pltpu.VMEM((128,), jnp.int32),
            pltpu.VMEM((128,), jnp.int32),
            pltpu.VMEM((2, D, CW), jnp.float32),
            pltpu.VMEM((D, TAIL), jnp.float32),
            pltpu.VMEM((RB, 128), jnp.float32),
            pltpu.VMEM((RB,), jnp.int32),
            pltpu.SemaphoreType.DMA((2,)),
            pltpu.SemaphoreType.DMA,
        ],
        compiler_params=pltpu.CompilerParams(
            needs_layout_passes=False, use_tc_tiling_on_sc=True),
    )
    def _k(users_hbm, items_hbm, utt_hbm, itt_hbm, utail_hbm, itail_hbm,
           urows_hbm, irows_hbm,
           idx_v, wl_v, wl2_v, counts_v, offs_v, cur_v,
           cbuf_v, tbuf_v, rbuf_v, blist_v, csem, fsem):
        cid = lax.axis_index("c")
        sid = lax.axis_index("s")

        def reset_blist():
            for t in range(RB // L):
                blist_v[pl.ds(t * L, L)] = B + t * L + _lanes()

        def flush(rows_hbm):
            pltpu.async_copy(rbuf_v, rows_hbm.at[blist_v], fsem).wait()
            reset_blist()

        def scan_bucket(ki, slot, rows_hbm, src_v, enable):
            lanes = _lanes()
            kf = jnp.full((L,), ki, jnp.int32)
            start = plsc.load_gather(offs_v, [kf])[0]
            cnt = jnp.where(enable, plsc.load_gather(counts_v, [kf])[0], 0)
            af = (start >> 4) << 4
            end = start + cnt
            nv = jnp.where(cnt > 0, (end - af + L - 1) >> 4, 0)

            def wl_body(v, slot):
                av = af + v * L
                evec = wl2_v[pl.ds(av, L)]
                pos = av + lanes
                m = (pos >= start) & (pos < end)
                col = evec & (CW - 1)
                bvec = lax.shift_right_logical(evec, 9)
                nhit = plsc.all_reduce_population_count(m)[0]
                mi = m.astype(jnp.int32)

                s = slot
                for k in range(L):
                    @pl.when(mi[k] != 0)
                    def _(s=s, k=k):
                        ck = jnp.full((L,), col[k], jnp.int32)
                        for q in range(D // L):
                            qv = plsc.load_gather(
                                src_v, [q * L + lanes, ck])
                            rbuf_v[s, pl.ds(q * L, L)] = qv
                        plsc.store_scatter(
                            blist_v, [jnp.full((L,), s, jnp.int32)],
                            jnp.full((L,), bvec[k], jnp.int32),
                            mask=lanes == 0)
                    s = s + mi[k]

                slot = slot + nhit
                need = slot >= FLUSH_AT

                @pl.when(need)
                def _():
                    flush(rows_hbm)

                return jnp.where(need, 0, slot)

            return lax.fori_loop(0, nv, wl_body, slot)

        def sweep(idx_hbm, tab_hbm, tail_hbm, rows_hbm):
            reset_blist()
            zeros = jnp.zeros((L,), jnp.int32)
            for t in range(128 // L):
                counts_v[pl.ds(t * L, L)] = zeros

            SEG = B // 4 + L
            SEGV = NV // 8
            totals = (jnp.int32(0),) * 4
            for r in range(2):
                pltpu.sync_copy(idx_hbm.at[pl.ds(r * (B // 2), B // 2)],
                                idx_v)

                def build(v, ws, r=r):
                    ws = list(ws)
                    for s in range(4):
                        bv = s * SEGV + v
                        bvec = r * (B // 2) + bv * L + _lanes()
                        uvec = idx_v[pl.ds(bv * L, L)]
                        kiv = lax.shift_right_logical(uvec, 13)
                        packed = ((kiv << 23) | ((uvec & (CW - 1)) << 14)
                                  | bvec)
                        m = ((uvec >> 9) & (NS - 1)) == sid
                        plsc.store_compressed(
                            wl_v.at[pl.ds(s * SEG + ws[s], L)],
                            packed, mask=m)
                        plsc.addupdate_scatter(
                            counts_v, [kiv],
                            jnp.ones((L,), jnp.int32), mask=m)
                        ws[s] = ws[s] + plsc.all_reduce_population_count(m)[0]
                    return tuple(ws)

                totals = lax.fori_loop(0, SEGV, build, totals)
            for s in range(4):
                wl_v[pl.ds(s * SEG + totals[s], L)] = jnp.zeros(
                    (L,), jnp.int32)

            carry = jnp.int32(0)
            for t in range(128 // L):
                c16 = counts_v[pl.ds(t * L, L)]
                cs = plsc.cumsum(c16)
                ex = cs - c16 + carry
                offs_v[pl.ds(t * L, L)] = ex
                cur_v[pl.ds(t * L, L)] = ex
                carry = carry + cs[L - 1]

            lanes = _lanes()

            for s in range(4):
                def reorder(w, carry, s=s):
                    wvec = wl_v[pl.ds(s * SEG + w * L, L)]
                    vi = ((w * L + lanes) < totals[s]).astype(jnp.int32)
                    kiv = lax.shift_right_logical(wvec, 23)
                    packed = ((lax.shift_right_logical(wvec, 14)
                               & (CW - 1))
                              | ((wvec & (2 ** 14 - 1)) << 9))
                    for k in range(L):
                        @pl.when(vi[k] != 0)
                        def _(k=k):
                            kk = jnp.full((L,), kiv[k], jnp.int32)
                            p = plsc.load_gather(cur_v, [kk])[0]
                            l0 = lanes == 0
                            plsc.store_scatter(
                                cur_v, [kk],
                                jnp.full((L,), p + 1, jnp.int32), mask=l0)
                            plsc.store_scatter(
                                wl2_v, [jnp.full((L,), p, jnp.int32)],
                                jnp.full((L,), packed[k], jnp.int32),
                                mask=l0)
                    return carry

                lax.fori_loop(0, (totals[s] + L - 1) // L, reorder, 0)

            def cslab(ki):
                return jnp.minimum(sid + ki * NS, NCHUNK_FULL - 1)

            def fire(ki, buf):
                c = cslab(ki)
                for g in range(D // 8):
                    pltpu.async_copy(
                        tab_hbm.at[pl.ds(g * 8, 8), pl.ds(c * CW, CW)],
                        cbuf_v.at[buf, pl.ds(g * 8, 8)], csem.at[buf])

            def wait(ki, buf):
                c = cslab(ki)
                for g in range(D // 8):
                    pltpu.make_async_copy(
                        tab_hbm.at[pl.ds(g * 8, 8), pl.ds(c * CW, CW)],
                        cbuf_v.at[buf, pl.ds(g * 8, 8)],
                        csem.at[buf]).wait()

            fire(0, 0)

            def pair_body(kp, slot):
                for par in range(2):
                    ki = kp * 2 + par
                    fire(ki + 1, (par + 1) % 2)
                    wait(ki, par)
                    slot = scan_bucket(ki, slot, rows_hbm, cbuf_v.at[par],
                                       sid + ki * NS < NCHUNK_FULL)
                return slot

            slot = lax.fori_loop(0, NK // 2, pair_body, 0)
            ki = NK - 1
            wait(ki, ki % 2)
            slot = scan_bucket(jnp.int32(ki), slot, rows_hbm,
                               cbuf_v.at[ki % 2],
                               sid + ki * NS < NCHUNK_FULL)

            if TAIL:
                pltpu.sync_copy(tail_hbm, tbuf_v)
                slot = scan_bucket(jnp.int32(NK - 1), slot, rows_hbm,
                                   tbuf_v, sid == NCHUNK_FULL % NS)

            flush(rows_hbm)

        @pl.when(cid == 0)
        def _():
            sweep(users_hbm, utt_hbm, utail_hbm, urows_hbm)

        @pl.when(cid == 1)
        def _():
            sweep(items_hbm, itt_hbm, itail_hbm, irows_hbm)

    return _k


def _make_score_kernel(B, D, NC, NS):
    NW = NC * NS
    bw = B // NW
    CH = 128
    nchunk = bw // CH
    nbuf = 2

    mesh = plsc.VectorSubcoreMesh(core_axis_name="c", subcore_axis_name="s")

    @functools.partial(
        pl.kernel,
        mesh=mesh,
        out_type=jax.ShapeDtypeStruct((B,), jnp.float32),
        scratch_types=[
            pltpu.VMEM((nchunk, CH), jnp.int32),
            pltpu.VMEM((nbuf, CH, 128), jnp.float32),
            pltpu.VMEM((nbuf, CH, 128), jnp.float32),
            pltpu.VMEM((bw,), jnp.float32),
            pltpu.VMEM((bw,), jnp.float32),
            pltpu.SemaphoreType.DMA,
            pltpu.SemaphoreType.DMA((2,)),
            pltpu.SemaphoreType.DMA,
        ],
        compiler_params=pltpu.CompilerParams(
            needs_layout_passes=False, use_tc_tiling_on_sc=True),
    )
    def _k(items_hbm, urows_hbm, irows_hbm, pop_hbm, out_hbm,
           iidx_v, ubuf_v, ibuf_v, pop_v, out_v, isem, gsem, psem):
        wid = lax.axis_index("s") * NC + lax.axis_index("c")
        base = wid * bw

        idx_copies = [
            pltpu.async_copy(items_hbm.at[pl.ds(base + j * CH, CH)],
                             iidx_v.at[j], isem)
            for j in range(nchunk)
        ]
        for c in idx_copies:
            c.wait()

        pop_copies = [
            pltpu.async_copy(pop_hbm.at[iidx_v.at[j]],
                             pop_v.at[pl.ds(j * CH, CH)], psem)
            for j in range(nchunk)
        ]

        def fire(j):
            buf = j % nbuf
            s = pl.ds(base + j * CH, CH)
            u = pltpu.async_copy(urows_hbm.at[s], ubuf_v.at[buf],
                                 gsem.at[buf])
            i = pltpu.async_copy(irows_hbm.at[s], ibuf_v.at[buf],
                                 gsem.at[buf])
            return (u, i)

        inflight = [fire(j) for j in range(min(nbuf, nchunk))]
        for c in pop_copies:
            c.wait()

        for j in range(nchunk):
            buf = j % nbuf
            for c in inflight[j]:
                c.wait()

            def group(g, carry):
                rows = g * L + _lanes()
                fl = pl.ds(j * CH + g * L, L)
                acc = jnp.zeros((L,), jnp.float32)
                for d in range(D):
                    cols = jnp.full((L,), d, jnp.int32)
                    uc = plsc.load_gather(ubuf_v.at[buf], [rows, cols])
                    ic = plsc.load_gather(ibuf_v.at[buf], [rows, cols])
                    acc = acc + uc * ic
                pop = pop_v[fl]
                r = jnp.where(acc > 0.0, acc + 1.0,
                              jnp.exp(jnp.minimum(acc, 0.0)))
                out_v[fl] = r * pop
                return carry

            lax.fori_loop(0, CH // L, group, 0)
            if j + nbuf < nchunk:
                inflight.append(fire(j + nbuf))

        pltpu.sync_copy(out_v, out_hbm.at[pl.ds(base, bw)])

    return _k


@jax.jit
def kernel(users, items, user_table, item_table, last_popularity):
    B = users.shape[0]
    V, D = user_table.shape
    info = plsc.get_sparse_core_info()
    NC, NS = info.num_cores, info.num_subcores
    sweep = _make_sweep_kernel(B, V, D, NC, NS)
    score = _make_score_kernel(B, D, NC, NS)
    users = users.astype(jnp.int32)
    items = items.astype(jnp.int32)
    utt = user_table.T
    itt = item_table.T
    ntail = V % 128
    utail = lax.slice(utt, (0, V - ntail), (D, V))
    itail = lax.slice(itt, (0, V - ntail), (D, V))
    rows_u, rows_i = sweep(users, items, utt, itt, utail, itail)
    return score(items, rows_u, rows_i, last_popularity)


# speedup vs baseline: 1.0379x; 1.0379x over previous
import functools

import jax
import jax.numpy as jnp
from jax import lax
from jax.experimental import pallas as pl
from jax.experimental.pallas import tpu as pltpu
from jax.experimental.pallas import tpu_sc as plsc

L = 16
CW = 512
RB = 64
FLUSH_AT = RB - L


def _lanes():
    return lax.iota(jnp.int32, L)


def _make_sweep_kernel(B, V, D, NC, NS):
    NCHUNK_FULL = V // CW
    TAIL = V - NCHUNK_FULL * CW
    NK = (NCHUNK_FULL + NS - 1) // NS
    NV = B // L
    RROWS = B + RB

    mesh = plsc.VectorSubcoreMesh(core_axis_name="c", subcore_axis_name="s")

    @functools.partial(
        pl.kernel,
        mesh=mesh,
        out_type=(jax.ShapeDtypeStruct((RROWS, 128), jnp.float32),
                  jax.ShapeDtypeStruct((RROWS, 128), jnp.float32)),
        scratch_types=[
            pltpu.VMEM((B // 2,), jnp.int32),
            pltpu.VMEM((B + 4 * L,), jnp.int32),
            pltpu.VMEM((B + L,), jnp.int32),
            pltpu.VMEM((128,), jnp.int32),
            pltpu.VMEM((128,), jnp.int32),
            pltpu.VMEM((128,), jnp.int32),
            pltpu.VMEM((512,), jnp.int32),
            pltpu.VMEM((2, D, CW), jnp.float32),
            pltpu.VMEM((D, TAIL), jnp.float32),
            pltpu.VMEM((RB, 128), jnp.float32),
            pltpu.VMEM((RB,), jnp.int32),
            pltpu.SemaphoreType.DMA((2,)),
            pltpu.SemaphoreType.DMA,
        ],
        compiler_params=pltpu.CompilerParams(
            needs_layout_passes=False, use_tc_tiling_on_sc=True),
    )
    def _k(users_hbm, items_hbm, utt_hbm, itt_hbm, utail_hbm, itail_hbm,
           urows_hbm, irows_hbm,
           idx_v, wl_v, wl2_v, counts_v, offs_v, cur_v, counts2_v,
           cbuf_v, tbuf_v, rbuf_v, blist_v, csem, fsem):
        cid = lax.axis_index("c")
        sid = lax.axis_index("s")

        def reset_blist():
            for t in range(RB // L):
                blist_v[pl.ds(t * L, L)] = B + t * L + _lanes()

        def flush(rows_hbm):
            pltpu.async_copy(rbuf_v, rows_hbm.at[blist_v], fsem).wait()
            reset_blist()

        def scan_bucket(ki, slot, rows_hbm, src_v, enable):
            lanes = _lanes()
            kf = jnp.full((L,), ki, jnp.int32)
            start = plsc.load_gather(offs_v, [kf])[0]
            cnt = jnp.where(enable, plsc.load_gather(counts_v, [kf])[0], 0)
            af = (start >> 4) << 4
            end = start + cnt
            nv = jnp.where(cnt > 0, (end - af + L - 1) >> 4, 0)

            def wl_body(v, slot):
                av = af + v * L
                evec = wl2_v[pl.ds(av, L)]
                pos = av + lanes
                m = (pos >= start) & (pos < end)
                col = evec & (CW - 1)
                bvec = lax.shift_right_logical(evec, 9)
                nhit = plsc.all_reduce_population_count(m)[0]
                mi = m.astype(jnp.int32)

                s = slot
                for k in range(L):
                    @pl.when(mi[k] != 0)
                    def _(s=s, k=k):
                        ck = jnp.full((L,), col[k], jnp.int32)
                        for q in range(D // L):
                            qv = plsc.load_gather(
                                src_v, [q * L + lanes, ck])
                            rbuf_v[s, pl.ds(q * L, L)] = qv
                        plsc.store_scatter(
                            blist_v, [jnp.full((L,), s, jnp.int32)],
                            jnp.full((L,), bvec[k], jnp.int32),
                            mask=lanes == 0)
                    s = s + mi[k]

                slot = slot + nhit
                need = slot >= FLUSH_AT

                @pl.when(need)
                def _():
                    flush(rows_hbm)

                return jnp.where(need, 0, slot)

            return lax.fori_loop(0, nv, wl_body, slot)

        def sweep(idx_hbm, tab_hbm, tail_hbm, rows_hbm):
            reset_blist()
            zeros = jnp.zeros((L,), jnp.int32)
            for t in range(128 // L):
                counts_v[pl.ds(t * L, L)] = zeros
            for t in range(512 // L):
                counts2_v[pl.ds(t * L, L)] = zeros

            SEG = B // 4 + L
            SEGV = NV // 8
            totals = (jnp.int32(0),) * 4
            for r in range(2):
                pltpu.sync_copy(idx_hbm.at[pl.ds(r * (B // 2), B // 2)],
                                idx_v)

                def build(v, ws, r=r):
                    ws = list(ws)
                    for s in range(4):
                        bv = s * SEGV + v
                        bvec = r * (B // 2) + bv * L + _lanes()
                        uvec = idx_v[pl.ds(bv * L, L)]
                        kiv = lax.shift_right_logical(uvec, 13)
                        packed = ((kiv << 23) | ((uvec & (CW - 1)) << 14)
                                  | bvec)
                        m = ((uvec >> 9) & (NS - 1)) == sid
                        plsc.store_compressed(
                            wl_v.at[pl.ds(s * SEG + ws[s], L)],
                            packed, mask=m)
                        ones = jnp.ones((L,), jnp.int32)
                        plsc.addupdate_scatter(counts_v, [kiv], ones, mask=m)
                        plsc.addupdate_scatter(
                            counts2_v,
                            [(kiv << 2) | (lax.shift_right_logical(uvec, 7)
                                           & 3)],
                            ones, mask=m)
                        ws[s] = ws[s] + plsc.all_reduce_population_count(m)[0]
                    return tuple(ws)

                totals = lax.fori_loop(0, SEGV, build, totals)
            for s in range(4):
                wl_v[pl.ds(s * SEG + totals[s], L)] = jnp.zeros(
                    (L,), jnp.int32)

            carry = jnp.int32(0)
            for t in range(128 // L):
                c16 = counts_v[pl.ds(t * L, L)]
                cs = plsc.cumsum(c16)
                ex = cs - c16 + carry
                offs_v[pl.ds(t * L, L)] = ex
                cur_v[pl.ds(t * L, L)] = ex
                carry = carry + cs[L - 1]

            lanes = _lanes()

            for s in range(4):
                def reorder(w, carry, s=s):
                    wvec = wl_v[pl.ds(s * SEG + w * L, L)]
                    vi = ((w * L + lanes) < totals[s]).astype(jnp.int32)
                    kiv = lax.shift_right_logical(wvec, 23)
                    packed = ((lax.shift_right_logical(wvec, 14)
                               & (CW - 1))
                              | ((wvec & (2 ** 14 - 1)) << 9))
                    for k in range(L):
                        @pl.when(vi[k] != 0)
                        def _(k=k):
                            kk = jnp.full((L,), kiv[k], jnp.int32)
                            p = plsc.load_gather(cur_v, [kk])[0]
                            l0 = lanes == 0
                            plsc.store_scatter(
                                cur_v, [kk],
                                jnp.full((L,), p + 1, jnp.int32), mask=l0)
                            plsc.store_scatter(
                                wl2_v, [jnp.full((L,), p, jnp.int32)],
                                jnp.full((L,), packed[k], jnp.int32),
                                mask=l0)
                    return carry

                lax.fori_loop(0, (totals[s] + L - 1) // L, reorder, 0)

            def cslab(ki):
                return jnp.minimum(sid + ki * NS, NCHUNK_FULL - 1)

            def blkcnt(ki, b):
                return plsc.load_gather(
                    counts2_v, [jnp.full((L,), (ki << 2) | b, jnp.int32)])[0]

            def fire(ki, buf):
                c = cslab(ki)
                for b in range(CW // 128):
                    @pl.when(blkcnt(ki, b) > 0)
                    def _(b=b):
                        pltpu.async_copy(
                            tab_hbm.at[:, pl.ds(c * CW + b * 128, 128)],
                            cbuf_v.at[buf, :, pl.ds(b * 128, 128)],
                            csem.at[buf])

            def wait(ki, buf):
                c = cslab(ki)
                for b in range(CW // 128):
                    @pl.when(blkcnt(ki, b) > 0)
                    def _(b=b):
                        pltpu.make_async_copy(
                            tab_hbm.at[:, pl.ds(c * CW + b * 128, 128)],
                            cbuf_v.at[buf, :, pl.ds(b * 128, 128)],
                            csem.at[buf]).wait()

            fire(0, 0)

            def pair_body(kp, slot):
                for par in range(2):
                    ki = kp * 2 + par
                    fire(ki + 1, (par + 1) % 2)
                    wait(ki, par)
                    slot = scan_bucket(ki, slot, rows_hbm, cbuf_v.at[par],
                                       sid + ki * NS < NCHUNK_FULL)
                return slot

            slot = lax.fori_loop(0, NK // 2, pair_body, 0)
            ki = NK - 1
            wait(ki, ki % 2)
            slot = scan_bucket(jnp.int32(ki), slot, rows_hbm,
                               cbuf_v.at[ki % 2],
                               sid + ki * NS < NCHUNK_FULL)

            if TAIL:
                pltpu.sync_copy(tail_hbm, tbuf_v)
                slot = scan_bucket(jnp.int32(NK - 1), slot, rows_hbm,
                                   tbuf_v, sid == NCHUNK_FULL % NS)

            flush(rows_hbm)

        @pl.when(cid == 0)
        def _():
            sweep(users_hbm, utt_hbm, utail_hbm, urows_hbm)

        @pl.when(cid == 1)
        def _():
            sweep(items_hbm, itt_hbm, itail_hbm, irows_hbm)

    return _k


def _make_score_kernel(B, D, NC, NS):
    NW = NC * NS
    bw = B // NW
    CH = 128
    nchunk = bw // CH
    nbuf = 2

    mesh = plsc.VectorSubcoreMesh(core_axis_name="c", subcore_axis_name="s")

    @functools.partial(
        pl.kernel,
        mesh=mesh,
        out_type=jax.ShapeDtypeStruct((B,), jnp.float32),
        scratch_types=[
            pltpu.VMEM((nchunk, CH), jnp.int32),
            pltpu.VMEM((nbuf, CH, 128), jnp.float32),
            pltpu.VMEM((nbuf, CH, 128), jnp.float32),
            pltpu.VMEM((bw,), jnp.float32),
            pltpu.VMEM((bw,), jnp.float32),
            pltpu.SemaphoreType.DMA,
            pltpu.SemaphoreType.DMA((2,)),
            pltpu.SemaphoreType.DMA,
        ],
        compiler_params=pltpu.CompilerParams(
            needs_layout_passes=False, use_tc_tiling_on_sc=True),
    )
    def _k(items_hbm, urows_hbm, irows_hbm, pop_hbm, out_hbm,
           iidx_v, ubuf_v, ibuf_v, pop_v, out_v, isem, gsem, psem):
        wid = lax.axis_index("s") * NC + lax.axis_index("c")
        base = wid * bw

        idx_copies = [
            pltpu.async_copy(items_hbm.at[pl.ds(base + j * CH, CH)],
                             iidx_v.at[j], isem)
            for j in range(nchunk)
        ]
        for c in idx_copies:
            c.wait()

        pop_copies = [
            pltpu.async_copy(pop_hbm.at[iidx_v.at[j]],
                             pop_v.at[pl.ds(j * CH, CH)], psem)
            for j in range(nchunk)
        ]

        def fire(j):
            buf = j % nbuf
            s = pl.ds(base + j * CH, CH)
            u = pltpu.async_copy(urows_hbm.at[s], ubuf_v.at[buf],
                                 gsem.at[buf])
            i = pltpu.async_copy(irows_hbm.at[s], ibuf_v.at[buf],
                                 gsem.at[buf])
            return (u, i)

        inflight = [fire(j) for j in range(min(nbuf, nchunk))]
        for c in pop_copies:
            c.wait()

        for j in range(nchunk):
            buf = j % nbuf
            for c in inflight[j]:
                c.wait()

            def group(g, carry):
                rows = g * L + _lanes()
                fl = pl.ds(j * CH + g * L, L)
                acc = jnp.zeros((L,), jnp.float32)
                for d in range(D):
                    cols = jnp.full((L,), d, jnp.int32)
                    uc = plsc.load_gather(ubuf_v.at[buf], [rows, cols])
                    ic = plsc.load_gather(ibuf_v.at[buf], [rows, cols])
                    acc = acc + uc * ic
                pop = pop_v[fl]
                r = jnp.where(acc > 0.0, acc + 1.0,
                              jnp.exp(jnp.minimum(acc, 0.0)))
                out_v[fl] = r * pop
                return carry

            lax.fori_loop(0, CH // L, group, 0)
            if j + nbuf < nchunk:
                inflight.append(fire(j + nbuf))

        pltpu.sync_copy(out_v, out_hbm.at[pl.ds(base, bw)])

    return _k


@jax.jit
def kernel(users, items, user_table, item_table, last_popularity):
    B = users.shape[0]
    V, D = user_table.shape
    info = plsc.get_sparse_core_info()
    NC, NS = info.num_cores, info.num_subcores
    sweep = _make_sweep_kernel(B, V, D, NC, NS)
    score = _make_score_kernel(B, D, NC, NS)
    users = users.astype(jnp.int32)
    items = items.astype(jnp.int32)
    utt = user_table.T
    itt = item_table.T
    ntail = V % 128
    utail = lax.slice(utt, (0, V - ntail), (D, V))
    itail = lax.slice(itt, (0, V - ntail), (D, V))
    rows_u, rows_i = sweep(users, items, utt, itt, utail, itail)
    return score(items, rows_u, rows_i, last_popularity)


# speedup vs baseline: 1.0554x; 1.0169x over previous
import functools

import jax
import jax.numpy as jnp
from jax import lax
from jax.experimental import pallas as pl
from jax.experimental.pallas import tpu as pltpu
from jax.experimental.pallas import tpu_sc as plsc

L = 16
CW = 512
RB = 48
FLUSH_AT = RB - L


def _lanes():
    return lax.iota(jnp.int32, L)


def _make_sweep_kernel(B, V, D, NC, NS):
    NCHUNK_FULL = V // CW
    TAIL = V - NCHUNK_FULL * CW
    NK = (NCHUNK_FULL + NS - 1) // NS
    NV = B // L
    RROWS = B + RB

    mesh = plsc.VectorSubcoreMesh(core_axis_name="c", subcore_axis_name="s")

    @functools.partial(
        pl.kernel,
        mesh=mesh,
        out_type=(jax.ShapeDtypeStruct((RROWS, 128), jnp.float32),
                  jax.ShapeDtypeStruct((RROWS, 128), jnp.float32)),
        scratch_types=[
            pltpu.VMEM((B // 2,), jnp.int32),
            pltpu.VMEM((B + 4 * L,), jnp.int32),
            pltpu.VMEM((B + L,), jnp.int32),
            pltpu.VMEM((128,), jnp.int32),
            pltpu.VMEM((128,), jnp.int32),
            pltpu.VMEM((128,), jnp.int32),
            pltpu.VMEM((512,), jnp.int32),
            pltpu.VMEM((2, D, CW), jnp.float32),
            pltpu.VMEM((D, TAIL), jnp.float32),
            pltpu.VMEM((2 * RB, 128), jnp.float32),
            pltpu.VMEM((2, RB), jnp.int32),
            pltpu.SemaphoreType.DMA((2,)),
            pltpu.SemaphoreType.DMA((2,)),
        ],
        compiler_params=pltpu.CompilerParams(
            needs_layout_passes=False, use_tc_tiling_on_sc=True),
    )
    def _k(users_hbm, items_hbm, utt_hbm, itt_hbm, utail_hbm, itail_hbm,
           urows_hbm, irows_hbm,
           idx_v, wl_v, wl2_v, counts_v, offs_v, cur_v, counts2_v,
           cbuf_v, tbuf_v, rbuf_v, blist_v, csem, fsem):
        cid = lax.axis_index("c")
        sid = lax.axis_index("s")

        def reset_blist(fb):
            for t in range(RB // L):
                blist_v[fb, pl.ds(t * L, L)] = B + t * L + _lanes()

        def flush_fire(fb, rows_hbm):
            o = pl.multiple_of(fb * RB, 8)
            pltpu.async_copy(rbuf_v.at[pl.ds(o, RB)],
                             rows_hbm.at[blist_v.at[fb]], fsem.at[fb])

        def flush_wait(fb, rows_hbm):
            o = pl.multiple_of(fb * RB, 8)
            pltpu.make_async_copy(rbuf_v.at[pl.ds(o, RB)],
                                  rows_hbm.at[blist_v.at[fb]],
                                  fsem.at[fb]).wait()

        def scan_bucket(ki, fstate, rows_hbm, src_v, enable):
            lanes = _lanes()
            kf = jnp.full((L,), ki, jnp.int32)
            start = plsc.load_gather(offs_v, [kf])[0]
            cnt = jnp.where(enable, plsc.load_gather(counts_v, [kf])[0], 0)
            af = (start >> 4) << 4
            end = start + cnt
            nv = jnp.where(cnt > 0, (end - af + L - 1) >> 4, 0)

            def wl_body(v, fstate):
                slot, fb, nf = fstate
                av = af + v * L
                evec = wl2_v[pl.ds(av, L)]
                pos = av + lanes
                m = (pos >= start) & (pos < end)
                col = evec & (CW - 1)
                bvec = lax.shift_right_logical(evec, 9)
                nhit = plsc.all_reduce_population_count(m)[0]
                mi = m.astype(jnp.int32)

                s = slot
                for k in range(L):
                    @pl.when(mi[k] != 0)
                    def _(s=s, k=k):
                        ck = jnp.full((L,), col[k], jnp.int32)
                        row = fb * RB + s
                        for q in range(D // L):
                            qv = plsc.load_gather(
                                src_v, [q * L + lanes, ck])
                            rbuf_v[row, pl.ds(q * L, L)] = qv
                        plsc.store_scatter(
                            blist_v,
                            [jnp.full((L,), fb, jnp.int32),
                             jnp.full((L,), s, jnp.int32)],
                            jnp.full((L,), bvec[k], jnp.int32),
                            mask=lanes == 0)
                    s = s + mi[k]

                slot = slot + nhit
                need = slot >= FLUSH_AT

                @pl.when(need)
                def _():
                    flush_fire(fb, rows_hbm)

                    @pl.when(nf >= 1)
                    def _():
                        flush_wait(1 - fb, rows_hbm)

                    reset_blist(1 - fb)

                slot = jnp.where(need, 0, slot)
                fb = jnp.where(need, 1 - fb, fb)
                nf = nf + jnp.where(need, 1, 0)
                return (slot, fb, nf)

            return lax.fori_loop(0, nv, wl_body, fstate)

        def sweep(idx_hbm, tab_hbm, tail_hbm, rows_hbm):
            reset_blist(0)
            reset_blist(1)
            zeros = jnp.zeros((L,), jnp.int32)
            for t in range(128 // L):
                counts_v[pl.ds(t * L, L)] = zeros
            for t in range(512 // L):
                counts2_v[pl.ds(t * L, L)] = zeros

            SEG = B // 4 + L
            SEGV = NV // 8
            totals = (jnp.int32(0),) * 4
            for r in range(2):
                pltpu.sync_copy(idx_hbm.at[pl.ds(r * (B // 2), B // 2)],
                                idx_v)

                def build(v, ws, r=r):
                    ws = list(ws)
                    for s in range(4):
                        bv = s * SEGV + v
                        bvec = r * (B // 2) + bv * L + _lanes()
                        uvec = idx_v[pl.ds(bv * L, L)]
                        kiv = lax.shift_right_logical(uvec, 13)
                        packed = ((kiv << 23) | ((uvec & (CW - 1)) << 14)
                                  | bvec)
                        m = ((uvec >> 9) & (NS - 1)) == sid
                        plsc.store_compressed(
                            wl_v.at[pl.ds(s * SEG + ws[s], L)],
                            packed, mask=m)
                        ones = jnp.ones((L,), jnp.int32)
                        plsc.addupdate_scatter(counts_v, [kiv], ones, mask=m)
                        plsc.addupdate_scatter(
                            counts2_v,
                            [(kiv << 2) | (lax.shift_right_logical(uvec, 7)
                                           & 3)],
                            ones, mask=m)
                        ws[s] = ws[s] + plsc.all_reduce_population_count(m)[0]
                    return tuple(ws)

                totals = lax.fori_loop(0, SEGV, build, totals)
            for s in range(4):
                wl_v[pl.ds(s * SEG + totals[s], L)] = jnp.zeros(
                    (L,), jnp.int32)

            carry = jnp.int32(0)
            for t in range(128 // L):
                c16 = counts_v[pl.ds(t * L, L)]
                cs = plsc.cumsum(c16)
                ex = cs - c16 + carry
                offs_v[pl.ds(t * L, L)] = ex
                cur_v[pl.ds(t * L, L)] = ex
                carry = carry + cs[L - 1]

            lanes = _lanes()

            for s in range(4):
                def reorder(w, carry, s=s):
                    wvec = wl_v[pl.ds(s * SEG + w * L, L)]
                    vi = ((w * L + lanes) < totals[s]).astype(jnp.int32)
                    kiv = lax.shift_right_logical(wvec, 23)
                    packed = ((lax.shift_right_logical(wvec, 14)
                               & (CW - 1))
                              | ((wvec & (2 ** 14 - 1)) << 9))
                    for k in range(L):
                        @pl.when(vi[k] != 0)
                        def _(k=k):
                            kk = jnp.full((L,), kiv[k], jnp.int32)
                            p = plsc.load_gather(cur_v, [kk])[0]
                            l0 = lanes == 0
                            plsc.store_scatter(
                                cur_v, [kk],
                                jnp.full((L,), p + 1, jnp.int32), mask=l0)
                            plsc.store_scatter(
                                wl2_v, [jnp.full((L,), p, jnp.int32)],
                                jnp.full((L,), packed[k], jnp.int32),
                                mask=l0)
                    return carry

                lax.fori_loop(0, (totals[s] + L - 1) // L, reorder, 0)

            def cslab(ki):
                return jnp.minimum(sid + ki * NS, NCHUNK_FULL - 1)

            def blkcnt(ki, b):
                return plsc.load_gather(
                    counts2_v, [jnp.full((L,), (ki << 2) | b, jnp.int32)])[0]

            def fire(ki, buf):
                c = cslab(ki)
                for b in range(CW // 128):
                    @pl.when(blkcnt(ki, b) > 0)
                    def _(b=b):
                        pltpu.async_copy(
                            tab_hbm.at[:, pl.ds(c * CW + b * 128, 128)],
                            cbuf_v.at[buf, :, pl.ds(b * 128, 128)],
                            csem.at[buf])

            def wait(ki, buf):
                c = cslab(ki)
                for b in range(CW // 128):
                    @pl.when(blkcnt(ki, b) > 0)
                    def _(b=b):
                        pltpu.make_async_copy(
                            tab_hbm.at[:, pl.ds(c * CW + b * 128, 128)],
                            cbuf_v.at[buf, :, pl.ds(b * 128, 128)],
                            csem.at[buf]).wait()

            fire(0, 0)

            def pair_body(kp, fstate):
                for par in range(2):
                    ki = kp * 2 + par
                    fire(ki + 1, (par + 1) % 2)
                    wait(ki, par)
                    fstate = scan_bucket(ki, fstate, rows_hbm,
                                         cbuf_v.at[par],
                                         sid + ki * NS < NCHUNK_FULL)
                return fstate

            fstate = (jnp.int32(0), jnp.int32(0), jnp.int32(0))
            fstate = lax.fori_loop(0, NK // 2, pair_body, fstate)
            ki = NK - 1
            wait(ki, ki % 2)
            fstate = scan_bucket(jnp.int32(ki), fstate, rows_hbm,
                                 cbuf_v.at[ki % 2],
                                 sid + ki * NS < NCHUNK_FULL)

            if TAIL:
                pltpu.sync_copy(tail_hbm, tbuf_v)
                fstate = scan_bucket(jnp.int32(NK - 1), fstate, rows_hbm,
                                     tbuf_v, sid == NCHUNK_FULL % NS)

            slot, fb, nf = fstate
            flush_fire(fb, rows_hbm)
            flush_wait(fb, rows_hbm)

            @pl.when(nf >= 1)
            def _():
                flush_wait(1 - fb, rows_hbm)

        @pl.when(cid == 0)
        def _():
            sweep(users_hbm, utt_hbm, utail_hbm, urows_hbm)

        @pl.when(cid == 1)
        def _():
            sweep(items_hbm, itt_hbm, itail_hbm, irows_hbm)

    return _k


def _make_score_kernel(B, D, NC, NS):
    NW = NC * NS
    bw = B // NW
    CH = 128
    nchunk = bw // CH
    nbuf = 2

    mesh = plsc.VectorSubcoreMesh(core_axis_name="c", subcore_axis_name="s")

    @functools.partial(
        pl.kernel,
        mesh=mesh,
        out_type=jax.ShapeDtypeStruct((B,), jnp.float32),
        scratch_types=[
            pltpu.VMEM((nchunk, CH), jnp.int32),
            pltpu.VMEM((nbuf, CH, 128), jnp.float32),
            pltpu.VMEM((nbuf, CH, 128), jnp.float32),
            pltpu.VMEM((bw,), jnp.float32),
            pltpu.VMEM((bw,), jnp.float32),
            pltpu.SemaphoreType.DMA,
            pltpu.SemaphoreType.DMA((2,)),
            pltpu.SemaphoreType.DMA,
        ],
        compiler_params=pltpu.CompilerParams(
            needs_layout_passes=False, use_tc_tiling_on_sc=True),
    )
    def _k(items_hbm, urows_hbm, irows_hbm, pop_hbm, out_hbm,
           iidx_v, ubuf_v, ibuf_v, pop_v, out_v, isem, gsem, psem):
        wid = lax.axis_index("s") * NC + lax.axis_index("c")
        base = wid * bw

        idx_copies = [
            pltpu.async_copy(items_hbm.at[pl.ds(base + j * CH, CH)],
                             iidx_v.at[j], isem)
            for j in range(nchunk)
        ]
        for c in idx_copies:
            c.wait()

        pop_copies = [
            pltpu.async_copy(pop_hbm.at[iidx_v.at[j]],
                             pop_v.at[pl.ds(j * CH, CH)], psem)
            for j in range(nchunk)
        ]

        def fire(j):
            buf = j % nbuf
            s = pl.ds(base + j * CH, CH)
            u = pltpu.async_copy(urows_hbm.at[s], ubuf_v.at[buf],
                                 gsem.at[buf])
            i = pltpu.async_copy(irows_hbm.at[s], ibuf_v.at[buf],
                                 gsem.at[buf])
            return (u, i)

        inflight = [fire(j) for j in range(min(nbuf, nchunk))]
        for c in pop_copies:
            c.wait()

        for j in range(nchunk):
            buf = j % nbuf
            for c in inflight[j]:
                c.wait()

            def group(g, carry):
                rows = g * L + _lanes()
                fl = pl.ds(j * CH + g * L, L)
                acc = jnp.zeros((L,), jnp.float32)
                for d in range(D):
                    cols = jnp.full((L,), d, jnp.int32)
                    uc = plsc.load_gather(ubuf_v.at[buf], [rows, cols])
                    ic = plsc.load_gather(ibuf_v.at[buf], [rows, cols])
                    acc = acc + uc * ic
                pop = pop_v[fl]
                r = jnp.where(acc > 0.0, acc + 1.0,
                              jnp.exp(jnp.minimum(acc, 0.0)))
                out_v[fl] = r * pop
                return carry

            lax.fori_loop(0, CH // L, group, 0)
            if j + nbuf < nchunk:
                inflight.append(fire(j + nbuf))

        pltpu.sync_copy(out_v, out_hbm.at[pl.ds(base, bw)])

    return _k


@jax.jit
def kernel(users, items, user_table, item_table, last_popularity):
    B = users.shape[0]
    V, D = user_table.shape
    info = plsc.get_sparse_core_info()
    NC, NS = info.num_cores, info.num_subcores
    sweep = _make_sweep_kernel(B, V, D, NC, NS)
    score = _make_score_kernel(B, D, NC, NS)
    users = users.astype(jnp.int32)
    items = items.astype(jnp.int32)
    utt = user_table.T
    itt = item_table.T
    ntail = V % 128
    utail = lax.slice(utt, (0, V - ntail), (D, V))
    itail = lax.slice(itt, (0, V - ntail), (D, V))
    rows_u, rows_i = sweep(users, items, utt, itt, utail, itail)
    return score(items, rows_u, rows_i, last_popularity)
